# baseline (speedup 1.0000x reference)
import functools

import jax
import jax.numpy as jnp
from jax import lax
from jax.experimental import pallas as pl
from jax.experimental.pallas import tpu as pltpu
from jax.experimental.pallas import tpu_sc as plsc

_FL = 512.0


def _build_sc_kernel(n_total, n_ch, hw, h, w, nb, n_workers, chunk):
    pw = n_total // n_workers
    nchunks = pw // chunk
    ngroups = chunk // 16
    wpb = n_workers // nb
    wpb_shift = wpb.bit_length() - 1
    assert (1 << wpb_shift) == wpb

    mesh = plsc.VectorSubcoreMesh(core_axis_name="c", subcore_axis_name="s")
    info = plsc.get_sparse_core_info()
    nc = info.num_cores

    @functools.partial(
        pl.kernel,
        out_type=jax.ShapeDtypeStruct((n_total, n_ch), jnp.float32),
        mesh=mesh,
        compiler_params=pltpu.CompilerParams(
            needs_layout_passes=False, use_tc_tiling_on_sc=False),
        scratch_types=[
            pltpu.VMEM((chunk * 3,), jnp.float32),
            pltpu.VMEM((192,), jnp.float32),
            pltpu.VMEM((chunk,), jnp.int32),
            pltpu.VMEM((chunk,), jnp.int32),
            pltpu.VMEM((chunk,), jnp.int32),
            pltpu.VMEM((chunk,), jnp.int32),
            pltpu.VMEM((chunk, 8), jnp.float32),
            pltpu.VMEM((chunk, 8), jnp.float32),
            pltpu.VMEM((chunk, 8), jnp.float32),
            pltpu.VMEM((chunk, 8), jnp.float32),
            pltpu.VMEM((chunk,), jnp.float32),
            pltpu.VMEM((chunk,), jnp.float32),
            pltpu.VMEM((chunk,), jnp.float32),
            pltpu.VMEM((chunk, n_ch), jnp.float32),
            pltpu.SemaphoreType.DMA,
        ],
    )
    def sc_kernel(locs_hbm, table_hbm, params_hbm, out_hbm,
                  locs_v, params_v, i00, i01, i10, i11,
                  r00, r01, r10, r11, fi_v, fj_v, zc_v, out_v, sem):
        wid = lax.axis_index("s") * nc + lax.axis_index("c")
        b = wid >> wpb_shift
        pbase = wid * pw
        tab_base = b * hw

        pltpu.sync_copy(params_hbm.at[pl.ds(b * 192, 192)], params_v)

        iota = lax.iota(jnp.int32, 16)

        def prm(j):
            return params_v[pl.ds(j * 16, 16)]

        m00, m10, m20 = prm(0), prm(3), prm(6)
        m01, m11, m21 = prm(1), prm(4), prm(7)
        m02, m12, m22 = prm(2), prm(5), prm(8)
        ppx, ppy, ppz = prm(9), prm(10), prm(11)

        def floor_i32(v):
            t = v.astype(jnp.int32)
            tf = t.astype(jnp.float32)
            return t - jnp.where(v < tf, 1, 0).astype(jnp.int32)

        def bf16_rne(v):
            r = lax.bitcast_convert_type(v, jnp.int32)
            r = r + 0x7FFF + ((r >> 16) & 1)
            return lax.bitcast_convert_type(
                r & jnp.int32(-65536), jnp.float32)

        def chunk_body(g, carry):
            goff = pbase + g * chunk
            pltpu.sync_copy(locs_hbm.at[pl.ds(goff * 3, chunk * 3)], locs_v)

            def phase_a(i, c2):
                o = i * 16
                p = o + iota
                p3 = p * 3
                x = bf16_rne(plsc.load_gather(locs_v, [p3]) - ppx)
                y = bf16_rne(plsc.load_gather(locs_v, [p3 + 1]) - ppy)
                z = bf16_rne(plsc.load_gather(locs_v, [p3 + 2]) - ppz)
                xc = x * m00 + y * m10 + z * m20
                yc = x * m01 + y * m11 + z * m21
                zc = x * m02 + y * m12 + z * m22
                zsafe = jnp.where(zc == 0.0, jnp.float32(1e-10), zc)
                px = xc * _FL / zsafe + w * 0.5
                py = yc * _FL / zsafe + h * 0.5
                fi = px - 0.5
                fj = py - 0.5
                x0i = floor_i32(fi)
                y0i = floor_i32(fj)
                x0c = jnp.minimum(jnp.maximum(x0i, 0), w - 1)
                x1c = jnp.minimum(jnp.maximum(x0i + 1, 0), w - 1)
                y0c = jnp.minimum(jnp.maximum(y0i, 0), h - 1)
                y1c = jnp.minimum(jnp.maximum(y0i + 1, 0), h - 1)
                yb0 = y0c * w + tab_base
                yb1 = y1c * w + tab_base
                i00[pl.ds(o, 16)] = yb0 + x0c
                i01[pl.ds(o, 16)] = yb0 + x1c
                i10[pl.ds(o, 16)] = yb1 + x0c
                i11[pl.ds(o, 16)] = yb1 + x1c
                fi_v[pl.ds(o, 16)] = fi
                fj_v[pl.ds(o, 16)] = fj
                zc_v[pl.ds(o, 16)] = zc
                return c2

            lax.fori_loop(0, ngroups, phase_a, 0, unroll=True)

            h00 = pltpu.async_copy(table_hbm.at[i00], r00, sem)
            h01 = pltpu.async_copy(table_hbm.at[i01], r01, sem)
            h10 = pltpu.async_copy(table_hbm.at[i10], r10, sem)
            h11 = pltpu.async_copy(table_hbm.at[i11], r11, sem)
            h00.wait()
            h01.wait()
            h10.wait()
            h11.wait()

            def phase_b(i, c2):
                o = i * 16
                p = o + iota
                zl = p >> 30
                zlo = p >> 31
                fi = fi_v[pl.ds(o, 16)]
                fj = fj_v[pl.ds(o, 16)]
                zc = zc_v[pl.ds(o, 16)]
                x0i = floor_i32(fi)
                y0i = floor_i32(fj)
                wx = fi - x0i.astype(jnp.float32)
                wy = fj - y0i.astype(jnp.float32)
                x1i = x0i + 1
                y1i = y0i + 1
                x0ok = (x0i >= 0) & (x0i < w)
                x1ok = (x1i >= 0) & (x1i < w)
                y0ok = (y0i >= 0) & (y0i < h)
                y1ok = (y1i >= 0) & (y1i < h)
                omwx = 1.0 - wx
                omwy = 1.0 - wy
                w00 = jnp.where(x0ok & y0ok, omwx * omwy, jnp.float32(0.0))
                w01 = jnp.where(x1ok & y0ok, wx * omwy, jnp.float32(0.0))
                w10 = jnp.where(x0ok & y1ok, omwx * wy, jnp.float32(0.0))
                w11 = jnp.where(x1ok & y1ok, wx * wy, jnp.float32(0.0))
                selx = (wx > 0.5) | ((wx == 0.5) & ((x0i & 1) == 1))
                sely = (wy > 0.5) | ((wy == 0.5) & ((y0i & 1) == 1))
                cd = zl + n_ch
                d00 = plsc.load_gather(r00, [p, cd])
                d01 = plsc.load_gather(r01, [p, cd])
                d10 = plsc.load_gather(r10, [p, cd])
                d11 = plsc.load_gather(r11, [p, cd])
                dm = jnp.where(sely, jnp.where(selx, d11, d10),
                               jnp.where(selx, d01, d00))
                px = fi + 0.5
                py = fj + 0.5
                vis = ((zc > 0.0) & (px >= 0.0) & (px < w)
                       & (py >= 0.0) & (py < h) & (zc <= dm))
                for c in range(n_ch):
                    cv = zl + c
                    v = (plsc.load_gather(r00, [p, cv]) * w00
                         + plsc.load_gather(r01, [p, cv]) * w01
                         + plsc.load_gather(r10, [p, cv]) * w10
                         + plsc.load_gather(r11, [p, cv]) * w11)
                    plsc.store_scatter(out_v, [p, zlo + c],
                                       jnp.where(vis, v, jnp.float32(0.0)))
                return c2

            lax.fori_loop(0, ngroups, phase_b, 0, unroll=True)

            pltpu.sync_copy(out_v, out_hbm.at[pl.ds(goff, chunk)])
            return carry

        lax.fori_loop(0, nchunks, chunk_body, 0)

    return sc_kernel


def _quat_to_mat(quat):
    qx, qy, qz, qw = quat[:, 0], quat[:, 1], quat[:, 2], quat[:, 3]
    qx2, qy2, qz2 = qx * qx, qy * qy, qz * qz
    qxqy, qxqz, qxqw = qx * qy, qx * qz, qx * qw
    qyqz, qyqw, qzqw = qy * qz, qy * qw, qz * qw
    r0 = jnp.stack([1 - 2 * qy2 - 2 * qz2, 2 * qxqy + 2 * qzqw,
                    2 * qxqz - 2 * qyqw], axis=1)
    r1 = jnp.stack([2 * qxqy - 2 * qzqw, 1 - 2 * qx2 - 2 * qz2,
                    2 * qyqz + 2 * qxqw], axis=1)
    r2 = jnp.stack([2 * qxqz + 2 * qyqw, 2 * qyqz - 2 * qxqw,
                    1 - 2 * qx2 - 2 * qy2], axis=1)
    return jnp.stack([r0, r1, r2], axis=1)


def kernel(locs, image, camera_pose, camera_rot, depth_mask):
    nb, n, _ = locs.shape
    _, n_ch, h, w = image.shape
    hw = h * w
    n_workers = 32
    chunk = 128
    wpb = n_workers // nb

    q = camera_rot / jnp.sqrt(jnp.sum(camera_rot ** 2, axis=1, keepdims=True))
    q = q * jnp.array([[-1.0, -1.0, -1.0, 1.0]], dtype=jnp.float32)
    rot = _quat_to_mat(q).astype(jnp.bfloat16).astype(jnp.float32)
    params = jnp.concatenate([rot.reshape(nb, 9), camera_pose], axis=1)
    params = jnp.broadcast_to(params[:, :, None], (nb, 12, 16)).reshape(-1)

    table = jnp.concatenate(
        [image.transpose(0, 2, 3, 1).reshape(nb, hw, n_ch),
         depth_mask.reshape(nb, hw, 1),
         jnp.zeros((nb, hw, 8 - n_ch - 1), jnp.float32)],
        axis=-1).reshape(nb * hw, 8)

    step = wpb * chunk
    npad = -(-n // step) * step
    locs_p = jnp.pad(locs, ((0, 0), (0, npad - n), (0, 0))).reshape(-1)

    sc = _build_sc_kernel(nb * npad, n_ch, hw, h, w, nb, n_workers, chunk)
    out = sc(locs_p, table, params)
    return out.reshape(nb, npad, n_ch)[:, :n, :]


# speedup vs baseline: 1.0502x; 1.0502x over previous
import functools

import jax
import jax.numpy as jnp
from jax import lax
from jax.experimental import pallas as pl
from jax.experimental.pallas import tpu as pltpu
from jax.experimental.pallas import tpu_sc as plsc

_FL = 512.0


def _build_sc_kernel(n_total, n_ch, hw, h, w, nb, n_workers, chunk):
    pw = n_total // n_workers
    nchunks = pw // chunk
    ngroups = chunk // 16
    k128 = chunk // 128
    wpb = n_workers // nb
    wpb_shift = wpb.bit_length() - 1
    assert (1 << wpb_shift) == wpb
    assert nchunks % 2 == 0

    mesh = plsc.VectorSubcoreMesh(core_axis_name="c", subcore_axis_name="s")
    info = plsc.get_sparse_core_info()
    nc = info.num_cores

    def set_types():
        return [
            pltpu.VMEM((chunk * 3,), jnp.float32),
            pltpu.VMEM((chunk,), jnp.int32),
            pltpu.VMEM((chunk,), jnp.int32),
            pltpu.VMEM((chunk,), jnp.int32),
            pltpu.VMEM((chunk,), jnp.int32),
            pltpu.VMEM((chunk, 8), jnp.float32),
            pltpu.VMEM((chunk, 8), jnp.float32),
            pltpu.VMEM((chunk, 8), jnp.float32),
            pltpu.VMEM((chunk, 8), jnp.float32),
            pltpu.VMEM((chunk,), jnp.float32),
            pltpu.VMEM((chunk,), jnp.float32),
            pltpu.VMEM((chunk,), jnp.float32),
            pltpu.VMEM((chunk, n_ch), jnp.float32),
            pltpu.SemaphoreType.DMA,
        ]

    @functools.partial(
        pl.kernel,
        out_type=jax.ShapeDtypeStruct((n_total, n_ch), jnp.float32),
        mesh=mesh,
        compiler_params=pltpu.CompilerParams(
            needs_layout_passes=False, use_tc_tiling_on_sc=False),
        scratch_types=[pltpu.VMEM((192,), jnp.float32)]
        + set_types() + set_types(),
    )
    def sc_kernel(locs_hbm, table_hbm, params_hbm, out_hbm, params_v,
                  *sets_flat):
        set0 = sets_flat[:14]
        set1 = sets_flat[14:]
        wid = lax.axis_index("s") * nc + lax.axis_index("c")
        b = wid >> wpb_shift
        pbase = wid * pw
        tab_base = b * hw

        pltpu.sync_copy(params_hbm.at[pl.ds(b * 192, 192)], params_v)

        iota = lax.iota(jnp.int32, 16)

        def prm(j):
            return params_v[pl.ds(j * 16, 16)]

        m00, m10, m20 = prm(0), prm(3), prm(6)
        m01, m11, m21 = prm(1), prm(4), prm(7)
        m02, m12, m22 = prm(2), prm(5), prm(8)
        ppx, ppy, ppz = prm(9), prm(10), prm(11)

        def floor_i32(v):
            t = v.astype(jnp.int32)
            tf = t.astype(jnp.float32)
            return t - jnp.where(v < tf, 1, 0).astype(jnp.int32)

        def bf16_rne(v):
            r = lax.bitcast_convert_type(v, jnp.int32)
            r = r + 0x7FFF + ((r >> 16) & 1)
            return lax.bitcast_convert_type(
                r & jnp.int32(-65536), jnp.float32)

        def prep(g, st):
            (locs_v, i00, i01, i10, i11, r00, r01, r10, r11,
             fi_v, fj_v, zc_v, out_v, sem) = st
            goff = pbase + g * chunk
            pltpu.sync_copy(locs_hbm.at[pl.ds(goff * 3, chunk * 3)], locs_v)

            def phase_a(i, c2):
                o = i * 16
                p = o + iota
                p3 = p * 3
                x = bf16_rne(plsc.load_gather(locs_v, [p3]) - ppx)
                y = bf16_rne(plsc.load_gather(locs_v, [p3 + 1]) - ppy)
                z = bf16_rne(plsc.load_gather(locs_v, [p3 + 2]) - ppz)
                xc = x * m00 + y * m10 + z * m20
                yc = x * m01 + y * m11 + z * m21
                zc = x * m02 + y * m12 + z * m22
                zsafe = jnp.where(zc == 0.0, jnp.float32(1e-10), zc)
                px = xc * _FL / zsafe + w * 0.5
                py = yc * _FL / zsafe + h * 0.5
                fi = px - 0.5
                fj = py - 0.5
                x0i = floor_i32(fi)
                y0i = floor_i32(fj)
                x0c = jnp.minimum(jnp.maximum(x0i, 0), w - 1)
                x1c = jnp.minimum(jnp.maximum(x0i + 1, 0), w - 1)
                y0c = jnp.minimum(jnp.maximum(y0i, 0), h - 1)
                y1c = jnp.minimum(jnp.maximum(y0i + 1, 0), h - 1)
                yb0 = y0c * w + tab_base
                yb1 = y1c * w + tab_base
                i00[pl.ds(o, 16)] = yb0 + x0c
                i01[pl.ds(o, 16)] = yb0 + x1c
                i10[pl.ds(o, 16)] = yb1 + x0c
                i11[pl.ds(o, 16)] = yb1 + x1c
                fi_v[pl.ds(o, 16)] = fi
                fj_v[pl.ds(o, 16)] = fj
                zc_v[pl.ds(o, 16)] = zc
                return c2

            lax.fori_loop(0, ngroups, phase_a, 0, unroll=2)
            pltpu.async_copy(table_hbm.at[i00], r00, sem)
            pltpu.async_copy(table_hbm.at[i01], r01, sem)
            pltpu.async_copy(table_hbm.at[i10], r10, sem)
            pltpu.async_copy(table_hbm.at[i11], r11, sem)

        def finish(g, st):
            (locs_v, i00, i01, i10, i11, r00, r01, r10, r11,
             fi_v, fj_v, zc_v, out_v, sem) = st
            goff = pbase + g * chunk
            pltpu.make_async_copy(table_hbm.at[i00], r00, sem).wait()
            pltpu.make_async_copy(table_hbm.at[i01], r01, sem).wait()
            pltpu.make_async_copy(table_hbm.at[i10], r10, sem).wait()
            pltpu.make_async_copy(table_hbm.at[i11], r11, sem).wait()

            def phase_b(i, c2):
                o = i * 16
                p = o + iota
                zl = p >> 30
                zlo = p >> 31
                fi = fi_v[pl.ds(o, 16)]
                fj = fj_v[pl.ds(o, 16)]
                zc = zc_v[pl.ds(o, 16)]
                x0i = floor_i32(fi)
                y0i = floor_i32(fj)
                wx = fi - x0i.astype(jnp.float32)
                wy = fj - y0i.astype(jnp.float32)
                x1i = x0i + 1
                y1i = y0i + 1
                x0ok = (x0i >= 0) & (x0i < w)
                x1ok = (x1i >= 0) & (x1i < w)
                y0ok = (y0i >= 0) & (y0i < h)
                y1ok = (y1i >= 0) & (y1i < h)
                omwx = 1.0 - wx
                omwy = 1.0 - wy
                w00 = jnp.where(x0ok & y0ok, omwx * omwy, jnp.float32(0.0))
                w01 = jnp.where(x1ok & y0ok, wx * omwy, jnp.float32(0.0))
                w10 = jnp.where(x0ok & y1ok, omwx * wy, jnp.float32(0.0))
                w11 = jnp.where(x1ok & y1ok, wx * wy, jnp.float32(0.0))
                selx = (wx > 0.5) | ((wx == 0.5) & ((x0i & 1) == 1))
                sely = (wy > 0.5) | ((wy == 0.5) & ((y0i & 1) == 1))
                cd = zl + n_ch
                d00 = plsc.load_gather(r00, [p, cd])
                d01 = plsc.load_gather(r01, [p, cd])
                d10 = plsc.load_gather(r10, [p, cd])
                d11 = plsc.load_gather(r11, [p, cd])
                dm = jnp.where(sely, jnp.where(selx, d11, d10),
                               jnp.where(selx, d01, d00))
                px = fi + 0.5
                py = fj + 0.5
                vis = ((zc > 0.0) & (px >= 0.0) & (px < w)
                       & (py >= 0.0) & (py < h) & (zc <= dm))
                for c in range(n_ch):
                    cv = zl + c
                    v = (plsc.load_gather(r00, [p, cv]) * w00
                         + plsc.load_gather(r01, [p, cv]) * w01
                         + plsc.load_gather(r10, [p, cv]) * w10
                         + plsc.load_gather(r11, [p, cv]) * w11)
                    plsc.store_scatter(out_v, [p, zlo + c],
                                       jnp.where(vis, v, jnp.float32(0.0)))
                return c2

            lax.fori_loop(0, ngroups, phase_b, 0, unroll=2)
            pltpu.sync_copy(out_v, out_hbm.at[pl.ds(goff, chunk)])

        ntp = nchunks // 2
        prep(0, set0)

        def body(t, carry):
            g0 = 2 * t
            prep(g0 + 1, set1)
            finish(g0, set0)

            @pl.when(t + 1 < ntp)
            def _():
                prep(g0 + 2, set0)

            finish(g0 + 1, set1)
            return carry

        lax.fori_loop(0, ntp, body, 0)

    return sc_kernel


def _quat_to_mat(quat):
    qx, qy, qz, qw = quat[:, 0], quat[:, 1], quat[:, 2], quat[:, 3]
    qx2, qy2, qz2 = qx * qx, qy * qy, qz * qz
    qxqy, qxqz, qxqw = qx * qy, qx * qz, qx * qw
    qyqz, qyqw, qzqw = qy * qz, qy * qw, qz * qw
    r0 = jnp.stack([1 - 2 * qy2 - 2 * qz2, 2 * qxqy + 2 * qzqw,
                    2 * qxqz - 2 * qyqw], axis=1)
    r1 = jnp.stack([2 * qxqy - 2 * qzqw, 1 - 2 * qx2 - 2 * qz2,
                    2 * qyqz + 2 * qxqw], axis=1)
    r2 = jnp.stack([2 * qxqz + 2 * qyqw, 2 * qyqz - 2 * qxqw,
                    1 - 2 * qx2 - 2 * qy2], axis=1)
    return jnp.stack([r0, r1, r2], axis=1)


def kernel(locs, image, camera_pose, camera_rot, depth_mask):
    nb, n, _ = locs.shape
    _, n_ch, h, w = image.shape
    hw = h * w
    n_workers = 32
    chunk = 512
    wpb = n_workers // nb

    q = camera_rot / jnp.sqrt(jnp.sum(camera_rot ** 2, axis=1, keepdims=True))
    q = q * jnp.array([[-1.0, -1.0, -1.0, 1.0]], dtype=jnp.float32)
    rot = _quat_to_mat(q).astype(jnp.bfloat16).astype(jnp.float32)
    params = jnp.concatenate([rot.reshape(nb, 9), camera_pose], axis=1)
    params = jnp.broadcast_to(params[:, :, None], (nb, 12, 16)).reshape(-1)

    table = jnp.concatenate(
        [image.transpose(0, 2, 3, 1).reshape(nb, hw, n_ch),
         depth_mask.reshape(nb, hw, 1),
         jnp.zeros((nb, hw, 8 - n_ch - 1), jnp.float32)],
        axis=-1).reshape(nb * hw, 8)

    step = wpb * chunk * 2
    npad = -(-n // step) * step
    locs_p = jnp.pad(locs, ((0, 0), (0, npad - n), (0, 0))).reshape(-1)

    sc = _build_sc_kernel(nb * npad, n_ch, hw, h, w, nb, n_workers, chunk)
    out = sc(locs_p, table, params)
    return out.reshape(nb, npad, n_ch)[:, :n, :]
